# Initial kernel scaffold; baseline (speedup 1.0000x reference)
import jax
import jax.numpy as jnp
from jax.experimental import pallas as pl


def kernel(exer_list, score_list, exer_conc_adj, exer_conc_w, conc_conc_w, exer_pote_w, lambd, guess, slide):
    raise NotImplementedError("write your pallas kernel here")


# speedup vs baseline: 3.3943x; 3.3943x over previous
import functools

import jax
import jax.numpy as jnp
from jax import lax
from jax.experimental import pallas as pl
from jax.experimental.pallas import tpu as pltpu
from jax.experimental.pallas import tpu_sc as plsc


def _colmax_body(epw_ref, m_ref):
    m_ref[...] = jnp.max(epw_ref[...], axis=0, keepdims=True)


def _build_body(ecw_ref, adj_ref, epw_ref, m_ref, g_ref):
    w = jax.nn.sigmoid(ecw_ref[...]) * adj_ref[...]
    pexp = jnp.exp(epw_ref[...] - m_ref[...])
    pad = g_ref.shape[1] - w.shape[1] - pexp.shape[1]
    z = jnp.zeros((w.shape[0], pad), jnp.float32)
    g_ref[...] = jnp.concatenate([w, pexp, z], axis=1)


def _make_sc_kernel(B, L, E, C, P, GW, NC, NS):
    NW = NC * NS
    SPW = B // NW
    CH0 = 104
    CH1 = L - CH0
    NV = C // 16
    NPV = P // 16
    CC = 128 // 16
    NG = L // 16
    REM = L - NG * 16
    mesh = plsc.VectorSubcoreMesh(core_axis_name="c", subcore_axis_name="s")
    f32 = jnp.float32

    @functools.partial(
        pl.kernel,
        out_type=(
            jax.ShapeDtypeStruct((B, C), f32),
            jax.ShapeDtypeStruct((B, C), f32),
            jax.ShapeDtypeStruct((B, P), f32),
            jax.ShapeDtypeStruct((B, P), f32),
        ),
        mesh=mesh,
        scratch_types=[
            pltpu.VMEM((L,), jnp.int32),
            pltpu.VMEM((L,), f32),
            pltpu.VMEM((L, GW), f32),
            pltpu.VMEM((C,), f32),
            pltpu.VMEM((C,), f32),
            pltpu.VMEM((P,), f32),
            pltpu.VMEM((P,), f32),
            pltpu.SemaphoreType.DMA,
            pltpu.SemaphoreType.DMA,
        ],
    )
    def sc_kernel(g_hbm, exer_hbm, score_hbm,
                  s_hbm, t_hbm, sp_hbm, tp_hbm,
                  idx_v, xs_v, grows,
                  sstage, tstage, spstage, tpstage,
                  sem0, sem1):
        wid = lax.axis_index("s") * NC + lax.axis_index("c")
        base = wid * SPW

        def student(j, carry):
            i = base + j
            pltpu.sync_copy(exer_hbm.at[i], idx_v)
            pltpu.sync_copy(score_hbm.at[i], xs_v)
            cp0 = pltpu.async_copy(
                g_hbm.at[idx_v.at[pl.ds(0, CH0)]], grows.at[pl.ds(0, CH0)], sem0)
            cp1 = pltpu.async_copy(
                g_hbm.at[idx_v.at[pl.ds(CH0, CH1)]], grows.at[pl.ds(CH0, CH1)], sem1)
            cp0.wait()
            cp1.wait()

            def accumulate(rows_ref, nvec, c0):
                def gbody(g, acc):
                    ss, tt = acc
                    l0 = g * 16
                    xv = xs_v[pl.ds(l0, 16)]
                    for j in range(16):
                        x = xv[j]
                        vs = [rows_ref[l0 + j, pl.ds(c0 + k * 16, 16)]
                              for k in range(nvec)]
                        ss = tuple(ss[k] + vs[k] for k in range(nvec))
                        tt = tuple(tt[k] + x * vs[k] for k in range(nvec))
                    return ss, tt

                z = tuple(jnp.zeros((16,), f32) for _ in range(nvec))
                ss, tt = lax.fori_loop(0, NG, gbody, (z, z))
                if REM:
                    xv = xs_v[pl.ds(L - 16, 16)]
                    ss, tt = list(ss), list(tt)
                    for j in range(REM):
                        x = xv[16 - REM + j]
                        vs = [rows_ref[NG * 16 + j, pl.ds(c0 + k * 16, 16)]
                              for k in range(nvec)]
                        for k in range(nvec):
                            ss[k] = ss[k] + vs[k]
                            tt[k] = tt[k] + x * vs[k]
                return ss, tt

            for cc in range(NV // CC):
                c0 = cc * 128
                ss, tt = accumulate(grows, CC, c0)
                for k in range(CC):
                    sstage[pl.ds(c0 + k * 16, 16)] = ss[k]
                    tstage[pl.ds(c0 + k * 16, 16)] = tt[k]

            pss, ptt = accumulate(grows, NPV, C)
            for k in range(NPV):
                spstage[pl.ds(k * 16, 16)] = pss[k]
                tpstage[pl.ds(k * 16, 16)] = ptt[k]

            pltpu.sync_copy(sstage, s_hbm.at[i])
            pltpu.sync_copy(tstage, t_hbm.at[i])
            pltpu.sync_copy(spstage, sp_hbm.at[i])
            pltpu.sync_copy(tpstage, tp_hbm.at[i])
            return carry

        lax.fori_loop(0, SPW, student, 0)

    return sc_kernel


def _mix_body(s_ref, t_ref, sp_ref, tp_ref, ccw_ref, a_ref, bm_ref):
    s = s_ref[...]
    t = t_ref[...]
    ew = jnp.exp(ccw_ref[...])
    nz = s != 0.0
    mask = nz.astype(jnp.float32)
    a1 = jnp.where(nz, t, 0.0) / jnp.where(nz, s, 1.0)
    num = lax.dot_general(a1, ew, (((1,), (0,)), ((), ())),
                          preferred_element_type=jnp.float32)
    den = lax.dot_general(mask, ew, (((1,), (0,)), ((), ())),
                          preferred_element_type=jnp.float32)
    a_ref[...] = num / den
    bm_ref[...] = tp_ref[...] / sp_ref[...]


def _y_body(w_ref, epw_ref, lam_ref, gue_ref, sli_ref, a_ref, bm_ref, y_ref):
    w = w_ref[...]
    rs = jnp.sum(w, axis=1, keepdims=True)
    w2 = w / jnp.maximum(rs, 1e-30)
    d2 = jax.nn.softmax(epw_ref[...], axis=1)
    ya = lax.dot_general(a_ref[...], w2, (((1,), (1,)), ((), ())),
                         preferred_element_type=jnp.float32)
    yb = lax.dot_general(bm_ref[...], d2, (((1,), (1,)), ((), ())),
                         preferred_element_type=jnp.float32)
    ls = jax.nn.sigmoid(lam_ref[...])
    sl = jax.nn.sigmoid(sli_ref[...])
    gu = jax.nn.sigmoid(gue_ref[...])
    ymid = (1.0 - ls) * ya + ls * yb
    ymid = jnp.clip(ymid, 1e-08, 1.0 - 1e-08)
    y_ref[...] = (1.0 - sl) * ymid + gu * (1.0 - ymid)


def kernel(exer_list, score_list, exer_conc_adj, exer_conc_w, conc_conc_w,
           exer_pote_w, lambd, guess, slide):
    B, L = exer_list.shape
    E, C = exer_conc_w.shape
    P = exer_pote_w.shape[1]
    f32 = jnp.float32
    exer = exer_list.astype(jnp.int32)

    m = pl.pallas_call(
        _colmax_body,
        out_shape=jax.ShapeDtypeStruct((1, P), f32),
    )(exer_pote_w)

    GW = C + 128
    EB1 = 1000
    G = pl.pallas_call(
        _build_body,
        grid=(E // EB1,),
        in_specs=[
            pl.BlockSpec((EB1, C), lambda i: (i, 0)),
            pl.BlockSpec((EB1, C), lambda i: (i, 0)),
            pl.BlockSpec((EB1, P), lambda i: (i, 0)),
            pl.BlockSpec((1, P), lambda i: (0, 0)),
        ],
        out_specs=pl.BlockSpec((EB1, GW), lambda i: (i, 0)),
        out_shape=jax.ShapeDtypeStruct((E, GW), f32),
    )(exer_conc_w, exer_conc_adj, exer_pote_w, m)

    info = plsc.get_sparse_core_info()
    NC, NS = info.num_cores, info.num_subcores
    sc = _make_sc_kernel(B, L, E, C, P, GW, NC, NS)
    S, T, SP, TP = sc(G, exer, score_list)

    A, Bm = pl.pallas_call(
        _mix_body,
        out_shape=[
            jax.ShapeDtypeStruct((B, C), f32),
            jax.ShapeDtypeStruct((B, P), f32),
        ],
    )(S, T, SP, TP, conc_conc_w)

    EB = 1024
    GE = (E + EB - 1) // EB
    Y = pl.pallas_call(
        _y_body,
        grid=(GE,),
        in_specs=[
            pl.BlockSpec((EB, C), lambda i: (i, 0)),
            pl.BlockSpec((EB, P), lambda i: (i, 0)),
            pl.BlockSpec((1, EB), lambda i: (0, i)),
            pl.BlockSpec((1, EB), lambda i: (0, i)),
            pl.BlockSpec((1, EB), lambda i: (0, i)),
            pl.BlockSpec((B, C), lambda i: (0, 0)),
            pl.BlockSpec((B, P), lambda i: (0, 0)),
        ],
        out_specs=pl.BlockSpec((B, EB), lambda i: (0, i)),
        out_shape=jax.ShapeDtypeStruct((B, E), f32),
    )(G, exer_pote_w, lambd, guess, slide, A, Bm)

    return A, Y
